# Initial kernel scaffold; baseline (speedup 1.0000x reference)
import jax
import jax.numpy as jnp
from jax.experimental import pallas as pl


def kernel(front_vec, front_dis, back_vec, back_dis, ske_mask):
    raise NotImplementedError("write your pallas kernel here")


# speedup vs baseline: 1.0223x; 1.0223x over previous
import jax
import jax.numpy as jnp
import numpy as np
from jax.experimental import pallas as pl
from jax.experimental.pallas import tpu as pltpu

_RES = 64
_EPS = 1e-6
_G = 8


def _kern(fv_ref, bv_ref, fd_ref, bd_ref, m_ref, out_ref):
    g = fv_ref.shape[0]
    fv2 = fv_ref[...].reshape(g * 20 * 32, 256)
    bv2 = bv_ref[...].reshape(g * 20 * 32, 256)
    m4 = m_ref[...]
    wf = m4 * fd_ref[...]
    wb = m4 * bd_ref[...]

    l_i = jax.lax.broadcasted_iota(jnp.int32, (256, 256), 0)
    o_i = jax.lax.broadcasted_iota(jnp.int32, (256, 256), 1)
    src_lane = jnp.where(o_i < 128, 2 * o_i, 2 * (o_i - 128) + 1)
    e_mat = jnp.where(l_i == src_lane, 1.0, 0.0)
    fmm = jnp.dot(fv2, e_mat, preferred_element_type=jnp.float32)
    bmm = jnp.dot(bv2, e_mat, preferred_element_type=jnp.float32)
    fe = fmm[:, :128].reshape(g, 20, 32, 128)
    fo = fmm[:, 128:].reshape(g, 20, 32, 128)
    be = bmm[:, :128].reshape(g, 20, 32, 128)
    bo = bmm[:, 128:].reshape(g, 20, 32, 128)

    t_i = jax.lax.broadcasted_iota(jnp.int32, (1, 1, 32, 128), 2)
    q_i = jax.lax.broadcasted_iota(jnp.int32, (1, 1, 32, 128), 3)
    yc = (2 * t_i + q_i // 64).astype(jnp.float32)
    xc = (q_i % 64).astype(jnp.float32)

    s_m = jnp.sum(m4, axis=(2, 3))
    s_my = jnp.sum(m4 * yc, axis=(2, 3))
    s_mx = jnp.sum(m4 * xc, axis=(2, 3))
    s_f0 = jnp.sum(wf * fe, axis=(2, 3))
    s_f1 = jnp.sum(wf * fo, axis=(2, 3))
    s_b0 = jnp.sum(wb * be, axis=(2, 3))
    s_b1 = jnp.sum(wb * bo, axis=(2, 3))

    denom = s_m + _EPS
    rcp = 1.0 / denom
    f0 = (s_f0 * float(_RES) + s_my) * rcp
    f1 = (s_f1 * float(_RES) + s_mx) * rcp
    b0 = (s_b0 * float(_RES) + s_my) * rcp
    b1 = (s_b1 * float(_RES) + s_mx) * rcp

    guard = s_m != 0.0
    gb0 = jnp.where(guard, b0, 0.0)
    gb1 = jnp.where(guard, b1, 0.0)
    k_i = jax.lax.broadcasted_iota(jnp.int32, (g, 20), 1)
    rmask = (k_i % 4 == 0).astype(jnp.float32)
    root_x = jnp.sum(gb0 * rmask, axis=-1, keepdims=True) * 0.2
    root_y = jnp.sum(gb1 * rmask, axis=-1, keepdims=True) * 0.2

    n_i = jax.lax.broadcasted_iota(jnp.int32, (g, 21), 1)
    nm1 = jnp.maximum(n_i - 1, 0)
    i4 = (nm1 // 4) * 4
    r = nm1 % 4
    idx_f = i4 + 3 - r
    idx_b = jnp.minimum(i4 + 4 - r, 19)
    c_f = jnp.where(r == 0, 1.0, 0.5)
    c_b = jnp.where(r == 0, 0.0, 0.5)

    tail_x = c_f * jnp.take_along_axis(f0, idx_f, axis=-1) + \
        c_b * jnp.take_along_axis(b0, idx_b, axis=-1)
    tail_y = c_f * jnp.take_along_axis(f1, idx_f, axis=-1) + \
        c_b * jnp.take_along_axis(b1, idx_b, axis=-1)

    lane0 = jax.lax.broadcasted_iota(jnp.int32, (g, 21), 1) == 0
    kp_x = jnp.where(lane0, root_x, tail_x) * 4.0
    kp_y = jnp.where(lane0, root_y, tail_y) * 4.0

    out_ref[...] = jnp.stack([kp_x, kp_y], axis=1)


@jax.jit
def kernel(front_vec, front_dis, back_vec, back_dis, ske_mask):
    b = front_vec.shape[0]
    fv = front_vec.reshape(b, 20, 32, 256)
    bv = back_vec.reshape(b, 20, 32, 256)
    fd = front_dis.reshape(b, 20, 32, 128)
    bd = back_dis.reshape(b, 20, 32, 128)
    m = ske_mask.reshape(b, 20, 32, 128)

    vspec = pl.BlockSpec((_G, 20, 32, 256), lambda i: (i, 0, 0, 0))
    wspec = pl.BlockSpec((_G, 20, 32, 128), lambda i: (i, 0, 0, 0))
    out = pl.pallas_call(
        _kern,
        grid=(b // _G,),
        in_specs=[vspec, vspec, wspec, wspec, wspec],
        out_specs=pl.BlockSpec((_G, 2, 21), lambda i: (i, 0, 0)),
        out_shape=jax.ShapeDtypeStruct((b, 2, 21), jnp.float32),
        compiler_params=pltpu.CompilerParams(
            dimension_semantics=("parallel",),
            vmem_limit_bytes=56 * 1024 * 1024,
        ),
        name="hourglass_kp_vote",
    )(fv, bv, fd, bd, m)
    return out.transpose(0, 2, 1)
